# baseline (speedup 1.0000x reference)
import functools
import jax
import jax.numpy as jnp
from jax import lax
from jax.experimental import pallas as pl
from jax.experimental.pallas import tpu as pltpu
from jax.experimental.pallas import tpu_sc as plsc

_ORDER = tuple(range(17))

_B, _N, _J, _D = 32, 243, 17, 256
_TOTAL = _B * _N * _J * _D
_NC, _NS = 2, 16
_NW = _NC * _NS
_PER_W = _TOTAL // _NW
_NCHUNK = 16
_CHUNK = _PER_W // _NCHUNK


def _make_sc_copy():
    nchunk = 32
    chunk = _PER_W // nchunk
    mesh = plsc.VectorSubcoreMesh(
        core_axis_name="c", subcore_axis_name="s",
        num_cores=_NC, num_subcores=_NS)

    @functools.partial(
        pl.kernel,
        mesh=mesh,
        out_type=jax.ShapeDtypeStruct((_TOTAL,), jnp.float32),
        scratch_types=[
            pltpu.VMEM((chunk,), jnp.float32),
            pltpu.VMEM((chunk,), jnp.float32),
            pltpu.SemaphoreType.DMA,
            pltpu.SemaphoreType.DMA,
            pltpu.SemaphoreType.DMA,
            pltpu.SemaphoreType.DMA,
        ],
    )
    def sc_copy(x_hbm, o_hbm, b0, b1, is0, is1, os0, os1):
        wid = lax.axis_index("s") * _NC + lax.axis_index("c")
        base = wid * _PER_W
        bufs = (b0, b1)
        isems = (is0, is1)
        osems = (os0, os1)

        def in_cp(g, k):
            return pltpu.make_async_copy(
                x_hbm.at[pl.ds(base + g * chunk, chunk)], bufs[k], isems[k])

        def out_cp(g, k):
            return pltpu.make_async_copy(
                bufs[k], o_hbm.at[pl.ds(base + g * chunk, chunk)], osems[k])

        in_cp(0, 0).start()
        for g in range(nchunk):
            k = g % 2
            nk = (g + 1) % 2
            if g + 1 < nchunk:
                if g >= 1:
                    out_cp(g - 1, nk).wait()
                in_cp(g + 1, nk).start()
            in_cp(g, k).wait()
            out_cp(g, k).start()
        out_cp(nchunk - 2, (nchunk - 2) % 2).wait()
        out_cp(nchunk - 1, (nchunk - 1) % 2).wait()

    return sc_copy


_SC_COPY_CACHE = []


def kernel(x):
    if not _SC_COPY_CACHE:
        _SC_COPY_CACHE.append(_make_sc_copy())
    flat = x.reshape(_TOTAL)
    out = _SC_COPY_CACHE[0](flat)
    return out.reshape(_B, _N, _J, _D)
